# baseline (speedup 1.0000x reference)
import functools

import jax
import jax.numpy as jnp
from jax import lax
from jax.experimental import pallas as pl
from jax.experimental.pallas import tpu as pltpu
from jax.experimental.pallas import tpu_sc as plsc

_VOCAB = 100000
_DIM = 64
_K = 4
_N = 16384

_NC = 2
_NS = 16
_NW = _NC * _NS
_BPW = _N // _NW
_LANE = 16

_XCH = 8
_XB = _BPW // _XCH
_FCH = 4
_FB = _BPW // _FCH
_MCH = 4
_MB = _BPW // _MCH

_MESH = plsc.VectorSubcoreMesh(core_axis_name="c", subcore_axis_name="s")
_PARAMS = pltpu.CompilerParams(use_tc_tiling_on_sc=True,
                               needs_layout_passes=False)


def _base():
    wid = lax.axis_index("s") * _NC + lax.axis_index("c")
    return wid * _BPW


@functools.partial(
    pl.kernel,
    mesh=_MESH,
    compiler_params=_PARAMS,
    out_type=jax.ShapeDtypeStruct((_N, _K * _DIM), jnp.float32),
    scratch_types=[
        pltpu.VMEM((_BPW,), jnp.int32),
        pltpu.VMEM((3, _XB, _K * _DIM), jnp.float32),
        pltpu.SemaphoreType.DMA,
        pltpu.SemaphoreType.DMA,
        pltpu.SemaphoreType.DMA,
        pltpu.SemaphoreType.DMA,
    ],
)
def _gather_x(ranks_hbm, mx_hbm, out_hbm, idx_v, x_v, s0, s1, s2, so):
    base = _base()
    pltpu.sync_copy(ranks_hbm.at[pl.ds(base, _BPW)], idx_v)
    sems = (s0, s1, s2)
    cp = [None] * _XCH
    ocp = [None] * _XCH
    for i in range(2):
        cp[i] = pltpu.async_copy(
            mx_hbm.at[idx_v.at[pl.ds(i * _XB, _XB)]], x_v.at[i], sems[i])
    for i in range(_XCH):
        cp[i].wait()
        if i + 2 < _XCH:
            if i >= 1:
                ocp[i - 1].wait()
            cp[i + 2] = pltpu.async_copy(
                mx_hbm.at[idx_v.at[pl.ds((i + 2) * _XB, _XB)]],
                x_v.at[(i + 2) % 3], sems[(i + 2) % 3])
        ocp[i] = pltpu.async_copy(
            x_v.at[i % 3], out_hbm.at[pl.ds(base + i * _XB, _XB)], so)
    for i in range(max(_XCH - 3, 0), _XCH):
        ocp[i].wait()


@functools.partial(
    pl.kernel,
    mesh=_MESH,
    compiler_params=_PARAMS,
    out_type=(
        jax.ShapeDtypeStruct((_DIM, _N), jnp.float32),
        jax.ShapeDtypeStruct((_K, _N), jnp.float32),
    ),
    scratch_types=[
        pltpu.VMEM((_BPW,), jnp.int32),
        pltpu.VMEM((2, _FB, 128), jnp.float32),
        pltpu.VMEM((_DIM, _FB), jnp.float32),
        pltpu.VMEM((_K, _FB), jnp.float32),
        pltpu.SemaphoreType.DMA,
        pltpu.SemaphoreType.DMA,
    ],
)
def _gather_fm(ranks_hbm, fm_hbm, out_f_hbm, out_m_hbm,
               idx_v, g_v, ft_v, mt_v, s0, s1):
    base = _base()
    pltpu.sync_copy(ranks_hbm.at[pl.ds(base, _BPW)], idx_v)
    sems = (s0, s1)
    cp = [None, None]
    cp[0] = pltpu.async_copy(
        fm_hbm.at[idx_v.at[pl.ds(0, _FB)]], g_v.at[0], sems[0])
    lanes = lax.iota(jnp.int32, _LANE)
    for i in range(_FCH):
        if i + 1 < _FCH:
            cp[(i + 1) % 2] = pltpu.async_copy(
                fm_hbm.at[idx_v.at[pl.ds((i + 1) * _FB, _FB)]],
                g_v.at[(i + 1) % 2], sems[(i + 1) % 2])
        cp[i % 2].wait()
        gbuf = g_v.at[i % 2]

        def _extf(k, _):
            rg = k // _DIM
            d = k % _DIM
            rows = lanes + rg * _LANE
            vals = plsc.load_gather(gbuf, [rows, lanes * 0 + d])
            plsc.store_scatter(ft_v, [lanes * 0 + d, rows], vals)
            return ()

        def _extm(k, _):
            rg = k // _K
            t = k % _K
            rows = lanes + rg * _LANE
            vals = plsc.load_gather(gbuf, [rows, lanes * 0 + (_DIM + t)])
            plsc.store_scatter(mt_v, [lanes * 0 + t, rows], vals)
            return ()

        lax.fori_loop(0, (_FB // _LANE) * _DIM, _extf, (), unroll=False)
        lax.fori_loop(0, (_FB // _LANE) * _K, _extm, (), unroll=False)
        pltpu.sync_copy(ft_v, out_f_hbm.at[:, pl.ds(base + i * _FB, _FB)])
        pltpu.sync_copy(mt_v, out_m_hbm.at[:, pl.ds(base + i * _FB, _FB)])


@jax.jit
def _fire_word(ranks, func_weight, measure_x, measure_m):
    mx_t = measure_x.transpose(1, 2, 0).reshape(_K * _DIM, _VOCAB)
    mx2 = mx_t.T
    fm_t = jnp.concatenate(
        [func_weight.T, measure_m.T,
         jnp.zeros((128 - _DIM - _K, _VOCAB), jnp.float32)], axis=0)
    fm = fm_t.T
    x_rows = _gather_x(ranks, mx2)
    f_t, m_t = _gather_fm(ranks, fm)
    return (f_t.T, x_rows.reshape(_N, _K, _DIM), m_t.T)


def kernel(ranks, func_weight, measure_x, measure_m):
    return _fire_word(ranks, func_weight, measure_x, measure_m)


# speedup vs baseline: 1.0002x; 1.0002x over previous
import functools

import jax
import jax.numpy as jnp
from jax import lax
from jax.experimental import pallas as pl
from jax.experimental.pallas import tpu as pltpu
from jax.experimental.pallas import tpu_sc as plsc

_VOCAB = 100000
_DIM = 64
_K = 4
_N = 16384

_NC = 2
_NS = 16
_NW = _NC * _NS
_BPW = _N // _NW
_LANE = 16

_XCH = 8
_XB = _BPW // _XCH
_FCH = 4
_FB = _BPW // _FCH

_MESH = plsc.VectorSubcoreMesh(core_axis_name="c", subcore_axis_name="s")
_PARAMS = pltpu.CompilerParams(use_tc_tiling_on_sc=True,
                               needs_layout_passes=False)


def _base():
    wid = lax.axis_index("s") * _NC + lax.axis_index("c")
    return wid * _BPW


@functools.partial(
    pl.kernel,
    mesh=_MESH,
    compiler_params=_PARAMS,
    out_type=jax.ShapeDtypeStruct((_N, _K * _DIM), jnp.float32),
    scratch_types=[
        pltpu.VMEM((_BPW,), jnp.int32),
        pltpu.VMEM((3, _XB, _K * _DIM), jnp.float32),
        pltpu.SemaphoreType.DMA,
        pltpu.SemaphoreType.DMA,
        pltpu.SemaphoreType.DMA,
        pltpu.SemaphoreType.DMA,
    ],
)
def _gather_x(ranks_hbm, mx_hbm, out_hbm, idx_v, x_v, s0, s1, s2, so):
    base = _base()
    pltpu.sync_copy(ranks_hbm.at[pl.ds(base, _BPW)], idx_v)
    sems = (s0, s1, s2)
    cp = [None] * _XCH
    ocp = [None] * _XCH
    for i in range(2):
        cp[i] = pltpu.async_copy(
            mx_hbm.at[idx_v.at[pl.ds(i * _XB, _XB)]], x_v.at[i], sems[i])
    for i in range(_XCH):
        cp[i].wait()
        if i + 2 < _XCH:
            if i >= 1:
                ocp[i - 1].wait()
            cp[i + 2] = pltpu.async_copy(
                mx_hbm.at[idx_v.at[pl.ds((i + 2) * _XB, _XB)]],
                x_v.at[(i + 2) % 3], sems[(i + 2) % 3])
        ocp[i] = pltpu.async_copy(
            x_v.at[i % 3], out_hbm.at[pl.ds(base + i * _XB, _XB)], so)
    for i in range(max(_XCH - 3, 0), _XCH):
        ocp[i].wait()


@functools.partial(
    pl.kernel,
    mesh=_MESH,
    compiler_params=_PARAMS,
    out_type=(
        jax.ShapeDtypeStruct((_DIM, _N), jnp.float32),
        jax.ShapeDtypeStruct((_K, _N), jnp.float32),
    ),
    scratch_types=[
        pltpu.VMEM((_BPW,), jnp.int32),
        pltpu.VMEM((2, _FB, 128), jnp.float32),
        pltpu.VMEM((_DIM, _FB), jnp.float32),
        pltpu.VMEM((_K, _FB), jnp.float32),
        pltpu.SemaphoreType.DMA,
        pltpu.SemaphoreType.DMA,
    ],
)
def _gather_fm(ranks_hbm, fm_hbm, out_f_hbm, out_m_hbm,
               idx_v, g_v, ft_v, mt_v, s0, s1):
    base = _base()
    pltpu.sync_copy(ranks_hbm.at[pl.ds(base, _BPW)], idx_v)
    sems = (s0, s1)
    cp = [None, None]
    cp[0] = pltpu.async_copy(
        fm_hbm.at[idx_v.at[pl.ds(0, _FB)]], g_v.at[0], sems[0])
    lanes = lax.iota(jnp.int32, _LANE)
    for i in range(_FCH):
        if i + 1 < _FCH:
            cp[(i + 1) % 2] = pltpu.async_copy(
                fm_hbm.at[idx_v.at[pl.ds((i + 1) * _FB, _FB)]],
                g_v.at[(i + 1) % 2], sems[(i + 1) % 2])
        cp[i % 2].wait()
        gbuf = g_v.at[i % 2]

        def _extf(k, _):
            rg = k // _DIM
            d = k % _DIM
            rows = lanes + rg * _LANE
            vals = plsc.load_gather(gbuf, [rows, lanes * 0 + d])
            plsc.store_scatter(ft_v, [lanes * 0 + d, rows], vals)
            return ()

        def _extm(k, _):
            rg = k // _K
            t = k % _K
            rows = lanes + rg * _LANE
            vals = plsc.load_gather(gbuf, [rows, lanes * 0 + (_DIM + t)])
            plsc.store_scatter(mt_v, [lanes * 0 + t, rows], vals)
            return ()

        lax.fori_loop(0, (_FB // _LANE) * _DIM, _extf, (), unroll=False)
        lax.fori_loop(0, (_FB // _LANE) * _K, _extm, (), unroll=False)
        pltpu.sync_copy(ft_v, out_f_hbm.at[:, pl.ds(base + i * _FB, _FB)])
        pltpu.sync_copy(mt_v, out_m_hbm.at[:, pl.ds(base + i * _FB, _FB)])


@jax.jit
def _fire_word(ranks, func_weight, measure_x, measure_m):
    mx_t = measure_x.transpose(1, 2, 0).reshape(_K * _DIM, _VOCAB)
    mx2 = mx_t.T
    fm_t = jnp.concatenate(
        [func_weight.T, measure_m.T,
         jnp.zeros((128 - _DIM - _K, _VOCAB), jnp.float32)], axis=0)
    fm = fm_t.T
    x_rows = _gather_x(ranks, mx2)
    f_t, m_t = _gather_fm(ranks, fm)
    return (f_t.T, x_rows.reshape(_N, _K, _DIM), m_t.T)


def kernel(ranks, func_weight, measure_x, measure_m):
    return _fire_word(ranks, func_weight, measure_x, measure_m)
